# Initial kernel scaffold; baseline (speedup 1.0000x reference)
import jax
import jax.numpy as jnp
from jax.experimental import pallas as pl


def kernel(uid, iid, niid, U, V, W, b, v_feat, mask, beta_u, beta_i):
    raise NotImplementedError("write your pallas kernel here")


# speedup vs baseline: 5.2414x; 5.2414x over previous
import functools

import jax
import jax.numpy as jnp
import numpy as np
from jax import lax
from jax.experimental import pallas as pl
from jax.experimental.pallas import tpu as pltpu
from jax.experimental.pallas import tpu_sc as plsc

USZ = 100000
ISZ = 50000
DIM = 64
FEAT_DIM = 64
RAW_FEAT = 128
BATCH = 4096
N_NEG = 50
W1, W2, W3, W4 = 1e-06, 1.0, 1.0, 1.0
WD1, WD2, WDI = 0.0001, 0.0001, 1.0
EMB = DIM + FEAT_DIM

NW = 32
BPW = BATCH // NW
NPW = BATCH * N_NEG // NW
CHUNK = 128
NCHUNK = NPW // CHUNK


RB = 1000
GA = ISZ // RB
URB = USZ // GA


def _stage_a_body(vf_ref, v_ref, u_ref, w_ref, b_ref, mask_ref, item_ref, reg_ref):
    i = pl.program_id(0)
    vf = vf_ref[...]
    nrm = jnp.sqrt(jnp.sum(vf * vf, axis=1, keepdims=True))
    vf = vf / jnp.maximum(nrm, 1e-12)
    vf = vf * mask_ref[...]
    feat = lax.dot_general(vf, w_ref[...], (((1,), (1,)), ((), ())),
                           preferred_element_type=jnp.float32)
    feat = feat + b_ref[...]
    v = v_ref[...]
    item_ref[...] = jnp.concatenate([v, feat], axis=1)
    u = u_ref[...]
    part = WD1 * (jnp.sum(u * u) + WDI * jnp.sum(v * v))

    @pl.when(i == 0)
    def _():
        w = w_ref[...]
        reg_ref[0, 0] = WD2 * (jnp.sum(w * w) + jnp.sum(b_ref[...] ** 2))

    reg_ref[0, 0] += part


_stage_a = pl.pallas_call(
    _stage_a_body,
    grid=(GA,),
    in_specs=[
        pl.BlockSpec((RB, RAW_FEAT), lambda i: (i, 0)),
        pl.BlockSpec((RB, DIM), lambda i: (i, 0)),
        pl.BlockSpec((URB, EMB), lambda i: (i, 0)),
        pl.BlockSpec((FEAT_DIM, RAW_FEAT), lambda i: (0, 0)),
        pl.BlockSpec((1, FEAT_DIM), lambda i: (0, 0)),
        pl.BlockSpec((1, RAW_FEAT), lambda i: (0, 0)),
    ],
    out_specs=[
        pl.BlockSpec((RB, EMB), lambda i: (i, 0)),
        pl.BlockSpec((1, 1), lambda i: (0, 0), memory_space=pltpu.SMEM),
    ],
    out_shape=[
        jax.ShapeDtypeStruct((ISZ, EMB), jnp.float32),
        jax.ShapeDtypeStruct((1, 1), jnp.float32),
    ],
)


_LB_MAP_NP = np.arange(NPW, dtype=np.int32) // N_NEG

_sc_mesh = plsc.VectorSubcoreMesh(core_axis_name="c", subcore_axis_name="s")


@functools.partial(
    pl.kernel,
    mesh=_sc_mesh,
    compiler_params=pltpu.CompilerParams(needs_layout_passes=False),
    out_type=[
        jax.ShapeDtypeStruct((BATCH,), jnp.float32),
        jax.ShapeDtypeStruct((BATCH,), jnp.float32),
        jax.ShapeDtypeStruct((BATCH * N_NEG,), jnp.float32),
        jax.ShapeDtypeStruct((BATCH * N_NEG,), jnp.float32),
    ],
    scratch_types=[
        pltpu.VMEM((BPW,), jnp.int32),
        pltpu.VMEM((BPW,), jnp.int32),
        pltpu.VMEM((BPW, EMB), jnp.float32),
        pltpu.VMEM((BPW, EMB), jnp.float32),
        pltpu.VMEM((BPW,), jnp.float32),
        pltpu.VMEM((BPW,), jnp.float32),
        pltpu.VMEM((BPW,), jnp.float32),
        pltpu.VMEM((BPW,), jnp.float32),
        pltpu.VMEM((CHUNK,), jnp.int32),
        pltpu.VMEM((CHUNK,), jnp.int32),
        pltpu.VMEM((CHUNK, EMB), jnp.float32),
        pltpu.VMEM((CHUNK,), jnp.float32),
        pltpu.VMEM((CHUNK,), jnp.float32),
        pltpu.VMEM((CHUNK,), jnp.float32),
        pltpu.SemaphoreType.DMA,
    ],
)
def _stage_b(uid_hbm, iid_hbm, niid_hbm, lbmap_hbm, u_hbm, item_hbm, bu_hbm,
             bi_hbm, pred_p_hbm, beta_p_hbm, pred_n_hbm, beta_n_hbm,
             uid_v, iid_v, urows, prows, bu_v, bi_v, pp_v, bp_v,
             nidx_v, lb_v, nrows, bin_v, pn_v, bn_v, sem):
    wid = lax.axis_index("s") * 2 + lax.axis_index("c")
    bbase = wid * BPW
    lane = lax.iota(jnp.int32, 16)

    pltpu.sync_copy(uid_hbm.at[pl.ds(bbase, BPW)], uid_v)
    pltpu.sync_copy(iid_hbm.at[pl.ds(bbase, BPW)], iid_v)
    pltpu.async_copy(u_hbm.at[uid_v], urows, sem).wait()
    pltpu.async_copy(item_hbm.at[iid_v], prows, sem).wait()
    pltpu.async_copy(bu_hbm.at[uid_v], bu_v, sem).wait()
    pltpu.async_copy(bi_hbm.at[iid_v], bi_v, sem).wait()

    def pos_group(g, carry):
        coll = jnp.zeros((16,), jnp.float32)
        for rr in range(16):
            r = g * 16 + rr
            acc = urows[r, pl.ds(0, 16)] * prows[r, pl.ds(0, 16)]
            for k in range(1, 8):
                acc = acc + urows[r, pl.ds(k * 16, 16)] * prows[r, pl.ds(k * 16, 16)]
            coll = jnp.where(lane == rr, jnp.sum(acc), coll)
        pp_v[pl.ds(g * 16, 16)] = coll
        return carry

    lax.fori_loop(0, BPW // 16, pos_group, 0)

    for g in range(BPW // 16):
        s = pl.ds(g * 16, 16)
        bp_v[s] = W1 + W2 * bu_v[s] * bi_v[s]

    pltpu.sync_copy(pp_v, pred_p_hbm.at[pl.ds(bbase, BPW)])
    pltpu.sync_copy(bp_v, beta_p_hbm.at[pl.ds(bbase, BPW)])

    def chunk_body(c, carry):
        nb = wid * NPW + c * CHUNK
        pltpu.sync_copy(niid_hbm.at[pl.ds(nb, CHUNK)], nidx_v)
        pltpu.sync_copy(lbmap_hbm.at[pl.ds(c * CHUNK, CHUNK)], lb_v)
        pltpu.async_copy(item_hbm.at[nidx_v], nrows, sem).wait()
        pltpu.async_copy(bi_hbm.at[nidx_v], bin_v, sem).wait()

        def neg_group(g, inner):
            coll = jnp.zeros((16,), jnp.float32)
            lbg = lb_v[pl.ds(g * 16, 16)]
            for rr in range(16):
                r = g * 16 + rr
                lb = lbg[rr]
                acc = nrows[r, pl.ds(0, 16)] * urows[lb, pl.ds(0, 16)]
                for k in range(1, 8):
                    acc = acc + nrows[r, pl.ds(k * 16, 16)] * urows[lb, pl.ds(k * 16, 16)]
                coll = jnp.where(lane == rr, jnp.sum(acc), coll)
            pn_v[pl.ds(g * 16, 16)] = coll
            return inner

        lax.fori_loop(0, CHUNK // 16, neg_group, 0)

        def bn_group(g, inner):
            s = pl.ds(g * 16, 16)
            buv = plsc.load_gather(bu_v, [lb_v[s]])
            bn_v[s] = W3 + W4 * buv * bin_v[s]
            return inner

        lax.fori_loop(0, CHUNK // 16, bn_group, 0)

        pltpu.sync_copy(pn_v, pred_n_hbm.at[pl.ds(nb, CHUNK)])
        pltpu.sync_copy(bn_v, beta_n_hbm.at[pl.ds(nb, CHUNK)])
        return carry

    lax.fori_loop(0, NCHUNK, chunk_body, 0)


PPR = 32
PNR = BATCH * N_NEG // 128


def _stage_c_body(pp_ref, bp_ref, pn_ref, bn_ref, reg_ref, out_ref):
    lp = jnp.sum(bp_ref[...] * jax.nn.softplus(-pp_ref[...]))
    ln = jnp.sum(bn_ref[...] * jax.nn.softplus(pn_ref[...])) * (1.0 / N_NEG)
    out_ref[0, 0] = lp + ln + reg_ref[0, 0]


_stage_c = pl.pallas_call(
    _stage_c_body,
    in_specs=[
        pl.BlockSpec((PPR, 128), lambda: (0, 0)),
        pl.BlockSpec((PPR, 128), lambda: (0, 0)),
        pl.BlockSpec((PNR, 128), lambda: (0, 0)),
        pl.BlockSpec((PNR, 128), lambda: (0, 0)),
        pl.BlockSpec((1, 1), lambda: (0, 0), memory_space=pltpu.SMEM),
    ],
    out_specs=pl.BlockSpec((1, 1), lambda: (0, 0), memory_space=pltpu.SMEM),
    out_shape=jax.ShapeDtypeStruct((1, 1), jnp.float32),
)


def kernel(uid, iid, niid, U, V, W, b, v_feat, mask, beta_u, beta_i):
    item_emb, regs = _stage_a(v_feat, V, U, W, b.reshape(1, FEAT_DIM),
                              mask.reshape(1, RAW_FEAT))
    pred_p, beta_p, pred_n, beta_n = _stage_b(
        uid, iid, niid.reshape(-1), jnp.asarray(_LB_MAP_NP), U, item_emb,
        beta_u, beta_i)
    loss = _stage_c(pred_p.reshape(PPR, 128), beta_p.reshape(PPR, 128),
                    pred_n.reshape(PNR, 128), beta_n.reshape(PNR, 128), regs)
    return loss[0, 0]


# speedup vs baseline: 9.7739x; 1.8648x over previous
import functools

import jax
import jax.numpy as jnp
import numpy as np
from jax import lax
from jax.experimental import pallas as pl
from jax.experimental.pallas import tpu as pltpu
from jax.experimental.pallas import tpu_sc as plsc

USZ = 100000
ISZ = 50000
DIM = 64
FEAT_DIM = 64
RAW_FEAT = 128
BATCH = 4096
N_NEG = 50
W1, W2, W3, W4 = 1e-06, 1.0, 1.0, 1.0
WD1, WD2, WDI = 0.0001, 0.0001, 1.0
EMB = DIM + FEAT_DIM

NW = 32
BPW = BATCH // NW
NPW = BATCH * N_NEG // NW
CHUNK = 128
NCHUNK = NPW // CHUNK


RB = 1000
GA = ISZ // RB
URB = USZ // GA


def _stage_a_body(vf_ref, v_ref, u_ref, w_ref, b_ref, mask_ref, item_ref, reg_ref):
    i = pl.program_id(0)
    vf = vf_ref[...]
    nrm = jnp.sqrt(jnp.sum(vf * vf, axis=1, keepdims=True))
    vf = vf / jnp.maximum(nrm, 1e-12)
    vf = vf * mask_ref[...]
    feat = lax.dot_general(vf, w_ref[...], (((1,), (1,)), ((), ())),
                           preferred_element_type=jnp.float32)
    feat = feat + b_ref[...]
    v = v_ref[...]
    item_ref[...] = jnp.concatenate([v, feat], axis=1)
    u = u_ref[...]
    part = WD1 * (jnp.sum(u * u) + WDI * jnp.sum(v * v))

    @pl.when(i == 0)
    def _():
        w = w_ref[...]
        reg_ref[0, 0] = WD2 * (jnp.sum(w * w) + jnp.sum(b_ref[...] ** 2))

    reg_ref[0, 0] += part


_stage_a = pl.pallas_call(
    _stage_a_body,
    grid=(GA,),
    in_specs=[
        pl.BlockSpec((RB, RAW_FEAT), lambda i: (i, 0)),
        pl.BlockSpec((RB, DIM), lambda i: (i, 0)),
        pl.BlockSpec((URB, EMB), lambda i: (i, 0)),
        pl.BlockSpec((FEAT_DIM, RAW_FEAT), lambda i: (0, 0)),
        pl.BlockSpec((1, FEAT_DIM), lambda i: (0, 0)),
        pl.BlockSpec((1, RAW_FEAT), lambda i: (0, 0)),
    ],
    out_specs=[
        pl.BlockSpec((RB, EMB), lambda i: (i, 0)),
        pl.BlockSpec((1, 1), lambda i: (0, 0), memory_space=pltpu.SMEM),
    ],
    out_shape=[
        jax.ShapeDtypeStruct((ISZ, EMB), jnp.float32),
        jax.ShapeDtypeStruct((1, 1), jnp.float32),
    ],
)


_LB_MAP_NP = np.arange(NPW, dtype=np.int32) // N_NEG

_sc_mesh = plsc.VectorSubcoreMesh(core_axis_name="c", subcore_axis_name="s")


@functools.partial(
    pl.kernel,
    mesh=_sc_mesh,
    compiler_params=pltpu.CompilerParams(needs_layout_passes=False),
    out_type=[
        jax.ShapeDtypeStruct((BATCH,), jnp.float32),
        jax.ShapeDtypeStruct((BATCH,), jnp.float32),
        jax.ShapeDtypeStruct((BATCH * N_NEG,), jnp.float32),
        jax.ShapeDtypeStruct((BATCH * N_NEG,), jnp.float32),
    ],
    scratch_types=[
        pltpu.VMEM((BPW,), jnp.int32),
        pltpu.VMEM((BPW,), jnp.int32),
        pltpu.VMEM((BPW, EMB), jnp.float32),
        pltpu.VMEM((BPW, EMB), jnp.float32),
        pltpu.VMEM((BPW,), jnp.float32),
        pltpu.VMEM((BPW,), jnp.float32),
        pltpu.VMEM((BPW,), jnp.float32),
        pltpu.VMEM((BPW,), jnp.float32),
        pltpu.VMEM((NPW,), jnp.int32),
        pltpu.VMEM((NPW,), jnp.int32),
        pltpu.VMEM((NPW,), jnp.float32),
        pltpu.VMEM((NPW,), jnp.float32),
        pltpu.VMEM((CHUNK,), jnp.int32),
        pltpu.VMEM((CHUNK,), jnp.int32),
        pltpu.VMEM((CHUNK, EMB), jnp.float32),
        pltpu.VMEM((CHUNK, EMB), jnp.float32),
        pltpu.VMEM((CHUNK,), jnp.float32),
        pltpu.VMEM((CHUNK,), jnp.float32),
        pltpu.SemaphoreType.DMA,
        pltpu.SemaphoreType.DMA,
        pltpu.SemaphoreType.DMA,
    ],
)
def _stage_b(uid_hbm, iid_hbm, niid_hbm, lbmap_hbm, u_hbm, item_hbm, bu_hbm,
             bi_hbm, pred_p_hbm, beta_p_hbm, pred_n_hbm, beta_n_hbm,
             uid_v, iid_v, urows, prows, bu_v, bi_v, pp_v, bp_v,
             nidx_all, lb_all, pn_all, bn_all,
             nidx_b0, nidx_b1, nrows_b0, nrows_b1, bin_b0, bin_b1,
             sem_pos, sem_b0, sem_b1):
    wid = lax.axis_index("s") * 2 + lax.axis_index("c")
    bbase = wid * BPW
    nbase = wid * NPW
    lane = lax.iota(jnp.int32, 16)
    bufs = ((nidx_b0, nrows_b0, bin_b0, sem_b0),
            (nidx_b1, nrows_b1, bin_b1, sem_b1))

    pltpu.sync_copy(uid_hbm.at[pl.ds(bbase, BPW)], uid_v)
    pltpu.sync_copy(iid_hbm.at[pl.ds(bbase, BPW)], iid_v)
    pltpu.async_copy(u_hbm.at[uid_v], urows, sem_pos)
    pltpu.async_copy(item_hbm.at[iid_v], prows, sem_pos)
    pltpu.async_copy(bu_hbm.at[uid_v], bu_v, sem_pos)
    pltpu.async_copy(bi_hbm.at[iid_v], bi_v, sem_pos)
    pltpu.sync_copy(niid_hbm.at[pl.ds(nbase, NPW)], nidx_all)
    pltpu.sync_copy(lbmap_hbm.at[pl.ds(0, NPW)], lb_all)

    def stage_idx(c, nidx_b):
        for g in range(CHUNK // 16):
            nidx_b[pl.ds(g * 16, 16)] = nidx_all[pl.ds(c * CHUNK + g * 16, 16)]

    def fire(c, buf):
        nidx_b, nrows_b, bin_b, sem = buf
        stage_idx(c, nidx_b)
        pltpu.async_copy(item_hbm.at[nidx_b], nrows_b, sem)
        pltpu.async_copy(bi_hbm.at[nidx_b], bin_b, sem)

    def drain(buf):
        nidx_b, nrows_b, bin_b, sem = buf
        pltpu.make_async_copy(item_hbm.at[nidx_b], nrows_b, sem).wait()
        pltpu.make_async_copy(bi_hbm.at[nidx_b], bin_b, sem).wait()

    def compute(c, buf):
        nidx_b, nrows_b, bin_b, sem = buf

        def neg_group(g, inner):
            coll = jnp.zeros((16,), jnp.float32)
            lbg = lb_all[pl.ds(c * CHUNK + g * 16, 16)]
            for rr in range(16):
                r = g * 16 + rr
                lb = lbg[rr]
                acc = nrows_b[r, pl.ds(0, 16)] * urows[lb, pl.ds(0, 16)]
                for k in range(1, 8):
                    acc = acc + nrows_b[r, pl.ds(k * 16, 16)] * urows[lb, pl.ds(k * 16, 16)]
                coll = jnp.where(lane == rr, jnp.sum(acc), coll)
            s = pl.ds(c * CHUNK + g * 16, 16)
            pn_all[s] = coll
            buv = plsc.load_gather(bu_v, [lbg])
            bn_all[s] = W3 + W4 * buv * bin_b[pl.ds(g * 16, 16)]
            return inner

        lax.fori_loop(0, CHUNK // 16, neg_group, 0)

    fire(0, bufs[0])

    pltpu.make_async_copy(u_hbm.at[uid_v], urows, sem_pos).wait()
    pltpu.make_async_copy(item_hbm.at[iid_v], prows, sem_pos).wait()
    pltpu.make_async_copy(bu_hbm.at[uid_v], bu_v, sem_pos).wait()
    pltpu.make_async_copy(bi_hbm.at[iid_v], bi_v, sem_pos).wait()

    def pos_group(g, carry):
        coll = jnp.zeros((16,), jnp.float32)
        for rr in range(16):
            r = g * 16 + rr
            acc = urows[r, pl.ds(0, 16)] * prows[r, pl.ds(0, 16)]
            for k in range(1, 8):
                acc = acc + urows[r, pl.ds(k * 16, 16)] * prows[r, pl.ds(k * 16, 16)]
            coll = jnp.where(lane == rr, jnp.sum(acc), coll)
        pp_v[pl.ds(g * 16, 16)] = coll
        return carry

    lax.fori_loop(0, BPW // 16, pos_group, 0)

    for g in range(BPW // 16):
        s = pl.ds(g * 16, 16)
        bp_v[s] = W1 + W2 * bu_v[s] * bi_v[s]

    pltpu.sync_copy(pp_v, pred_p_hbm.at[pl.ds(bbase, BPW)])
    pltpu.sync_copy(bp_v, beta_p_hbm.at[pl.ds(bbase, BPW)])

    def pair_body(i, carry):
        c = i * 2
        fire(c + 1, bufs[1])
        drain(bufs[0])
        compute(c, bufs[0])

        @pl.when(c + 2 < NCHUNK)
        def _():
            fire(c + 2, bufs[0])

        drain(bufs[1])
        compute(c + 1, bufs[1])
        return carry

    lax.fori_loop(0, NCHUNK // 2, pair_body, 0)

    pltpu.sync_copy(pn_all, pred_n_hbm.at[pl.ds(nbase, NPW)])
    pltpu.sync_copy(bn_all, beta_n_hbm.at[pl.ds(nbase, NPW)])


PPR = 32
PNR = BATCH * N_NEG // 128


def _stage_c_body(pp_ref, bp_ref, pn_ref, bn_ref, reg_ref, out_ref):
    lp = jnp.sum(bp_ref[...] * jax.nn.softplus(-pp_ref[...]))
    ln = jnp.sum(bn_ref[...] * jax.nn.softplus(pn_ref[...])) * (1.0 / N_NEG)
    out_ref[0, 0] = lp + ln + reg_ref[0, 0]


_stage_c = pl.pallas_call(
    _stage_c_body,
    in_specs=[
        pl.BlockSpec((PPR, 128), lambda: (0, 0)),
        pl.BlockSpec((PPR, 128), lambda: (0, 0)),
        pl.BlockSpec((PNR, 128), lambda: (0, 0)),
        pl.BlockSpec((PNR, 128), lambda: (0, 0)),
        pl.BlockSpec((1, 1), lambda: (0, 0), memory_space=pltpu.SMEM),
    ],
    out_specs=pl.BlockSpec((1, 1), lambda: (0, 0), memory_space=pltpu.SMEM),
    out_shape=jax.ShapeDtypeStruct((1, 1), jnp.float32),
)


def kernel(uid, iid, niid, U, V, W, b, v_feat, mask, beta_u, beta_i):
    item_emb, regs = _stage_a(v_feat, V, U, W, b.reshape(1, FEAT_DIM),
                              mask.reshape(1, RAW_FEAT))
    pred_p, beta_p, pred_n, beta_n = _stage_b(
        uid, iid, niid.reshape(-1), jnp.asarray(_LB_MAP_NP), U, item_emb,
        beta_u, beta_i)
    loss = _stage_c(pred_p.reshape(PPR, 128), beta_p.reshape(PPR, 128),
                    pred_n.reshape(PNR, 128), beta_n.reshape(PNR, 128), regs)
    return loss[0, 0]


# speedup vs baseline: 12.7559x; 1.3051x over previous
import functools

import jax
import jax.numpy as jnp
import numpy as np
from jax import lax
from jax.experimental import pallas as pl
from jax.experimental.pallas import tpu as pltpu
from jax.experimental.pallas import tpu_sc as plsc

USZ = 100000
ISZ = 50000
DIM = 64
FEAT_DIM = 64
RAW_FEAT = 128
BATCH = 4096
N_NEG = 50
W1, W2, W3, W4 = 1e-06, 1.0, 1.0, 1.0
WD1, WD2, WDI = 0.0001, 0.0001, 1.0
EMB = DIM + FEAT_DIM

NW = 32
BPW = BATCH // NW
NPW = BATCH * N_NEG // NW
CHUNK = 128
NCHUNK = NPW // CHUNK


RB = 1000
GA = ISZ // RB
URB = USZ // GA


def _stage_a_body(vf_ref, v_ref, u_ref, w_ref, b_ref, mask_ref, item_ref, reg_ref):
    i = pl.program_id(0)
    vf = vf_ref[...]
    nrm = jnp.sqrt(jnp.sum(vf * vf, axis=1, keepdims=True))
    vf = vf / jnp.maximum(nrm, 1e-12)
    vf = vf * mask_ref[...]
    feat = lax.dot_general(vf, w_ref[...], (((1,), (1,)), ((), ())),
                           preferred_element_type=jnp.float32)
    feat = feat + b_ref[...]
    v = v_ref[...]
    item_ref[...] = jnp.concatenate([v, feat], axis=1)
    u = u_ref[...]
    part = WD1 * (jnp.sum(u * u) + WDI * jnp.sum(v * v))

    @pl.when(i == 0)
    def _():
        w = w_ref[...]
        reg_ref[0, 0] = WD2 * (jnp.sum(w * w) + jnp.sum(b_ref[...] ** 2))

    reg_ref[0, 0] += part


_stage_a = pl.pallas_call(
    _stage_a_body,
    grid=(GA,),
    in_specs=[
        pl.BlockSpec((RB, RAW_FEAT), lambda i: (i, 0)),
        pl.BlockSpec((RB, DIM), lambda i: (i, 0)),
        pl.BlockSpec((URB, EMB), lambda i: (i, 0)),
        pl.BlockSpec((FEAT_DIM, RAW_FEAT), lambda i: (0, 0)),
        pl.BlockSpec((1, FEAT_DIM), lambda i: (0, 0)),
        pl.BlockSpec((1, RAW_FEAT), lambda i: (0, 0)),
    ],
    out_specs=[
        pl.BlockSpec((RB, EMB), lambda i: (i, 0)),
        pl.BlockSpec((1, 1), lambda i: (0, 0), memory_space=pltpu.SMEM),
    ],
    out_shape=[
        jax.ShapeDtypeStruct((ISZ, EMB), jnp.float32),
        jax.ShapeDtypeStruct((1, 1), jnp.float32),
    ],
)


LOG2E = 1.4426950408889634
LN2 = 0.6931471805599453

_sc_mesh = plsc.VectorSubcoreMesh(core_axis_name="c", subcore_axis_name="s")

NR_CH = 4
CH_NEG = NR_CH * N_NEG
NCH = BPW // NR_CH


def _softplus16(x):
    t = jnp.exp(-jnp.abs(x))
    z = t / (2.0 + t)
    z2 = z * z
    s = z * (2.0 + z2 * (2.0 / 3.0 + z2 * (2.0 / 5.0 + z2 * (
        2.0 / 7.0 + z2 * (2.0 / 9.0 + z2 * (2.0 / 11.0))))))
    return jnp.maximum(x, 0.0) + s


@functools.partial(
    pl.kernel,
    mesh=_sc_mesh,
    compiler_params=pltpu.CompilerParams(needs_layout_passes=False),
    out_type=[
        jax.ShapeDtypeStruct((NW * 16,), jnp.float32),
    ],
    scratch_types=[
        pltpu.VMEM((BPW,), jnp.int32),
        pltpu.VMEM((BPW,), jnp.int32),
        pltpu.VMEM((BPW, EMB), jnp.float32),
        pltpu.VMEM((BPW, EMB), jnp.float32),
        pltpu.VMEM((BPW,), jnp.float32),
        pltpu.VMEM((BPW,), jnp.float32),
        pltpu.VMEM((NPW,), jnp.int32),
        pltpu.VMEM((CH_NEG, EMB), jnp.float32),
        pltpu.VMEM((CH_NEG, EMB), jnp.float32),
        pltpu.VMEM((CH_NEG,), jnp.float32),
        pltpu.VMEM((CH_NEG,), jnp.float32),
        pltpu.VMEM((16,), jnp.float32),
        pltpu.SemaphoreType.DMA,
        pltpu.SemaphoreType.DMA,
        pltpu.SemaphoreType.DMA,
    ],
)
def _stage_b(uid_hbm, iid_hbm, niid_hbm, u_hbm, item_hbm, bu_hbm, bi_hbm,
             loss_hbm,
             uid_v, iid_v, urows, prows, bu_v, bi_v,
             nidx_all, nrows_b0, nrows_b1, bin_b0, bin_b1, loss_v,
             sem_pos, sem_b0, sem_b1):
    wid = lax.axis_index("s") * 2 + lax.axis_index("c")
    bbase = wid * BPW
    nbase = wid * NPW
    lane = lax.iota(jnp.int32, 16)
    bufs = ((nrows_b0, bin_b0, sem_b0), (nrows_b1, bin_b1, sem_b1))

    pltpu.sync_copy(uid_hbm.at[pl.ds(bbase, BPW)], uid_v)
    pltpu.sync_copy(iid_hbm.at[pl.ds(bbase, BPW)], iid_v)
    pltpu.async_copy(u_hbm.at[uid_v], urows, sem_pos)
    pltpu.async_copy(item_hbm.at[iid_v], prows, sem_pos)
    pltpu.async_copy(bu_hbm.at[uid_v], bu_v, sem_pos)
    pltpu.async_copy(bi_hbm.at[iid_v], bi_v, sem_pos)
    pltpu.sync_copy(niid_hbm.at[pl.ds(nbase, NPW)], nidx_all)

    def fire(c, buf):
        nrows_b, bin_b, sem = buf
        idx = nidx_all.at[pl.ds(c * CH_NEG, CH_NEG)]
        pltpu.async_copy(item_hbm.at[idx], nrows_b, sem)
        pltpu.async_copy(bi_hbm.at[idx], bin_b, sem)

    def drain(c, buf):
        nrows_b, bin_b, sem = buf
        idx = nidx_all.at[pl.ds(c * CH_NEG, CH_NEG)]
        pltpu.make_async_copy(item_hbm.at[idx], nrows_b, sem).wait()
        pltpu.make_async_copy(bi_hbm.at[idx], bin_b, sem).wait()

    def row_neg(c, row, buf, acc):
        nrows_b, bin_b, _ = buf
        r = c * NR_CH + row
        base = row * N_NEG
        uk = [urows[r, pl.ds(k * 16, 16)] for k in range(8)]
        bu_bc = plsc.load_gather(bu_v, [jnp.zeros((16,), jnp.int32) + r])
        for start, lo in ((0, 0), (16, 0), (32, 0), (34, 14)):
            coll = jnp.zeros((16,), jnp.float32)
            for jj in range(lo, 16):
                j = base + start + jj
                a = nrows_b[j, pl.ds(0, 16)] * uk[0]
                for k in range(1, 8):
                    a = a + nrows_b[j, pl.ds(k * 16, 16)] * uk[k]
                coll = jnp.where(lane == jj, jnp.sum(a), coll)
            binv = plsc.load_gather(bin_b, [base + start + lane])
            con = (W3 + W4 * bu_bc * binv) * _softplus16(coll)
            if lo:
                con = jnp.where(lane >= lo, con, 0.0)
            acc = acc + con
        return acc

    def chunk_neg(c, buf, acc):
        def rb(row, a):
            return row_neg(c, row, buf, a)
        return lax.fori_loop(0, NR_CH, rb, acc)

    fire(0, bufs[0])

    pltpu.make_async_copy(u_hbm.at[uid_v], urows, sem_pos).wait()
    pltpu.make_async_copy(item_hbm.at[iid_v], prows, sem_pos).wait()
    pltpu.make_async_copy(bu_hbm.at[uid_v], bu_v, sem_pos).wait()
    pltpu.make_async_copy(bi_hbm.at[iid_v], bi_v, sem_pos).wait()

    def pos_group(g, acc):
        coll = jnp.zeros((16,), jnp.float32)
        for rr in range(16):
            r = g * 16 + rr
            a = urows[r, pl.ds(0, 16)] * prows[r, pl.ds(0, 16)]
            for k in range(1, 8):
                a = a + urows[r, pl.ds(k * 16, 16)] * prows[r, pl.ds(k * 16, 16)]
            coll = jnp.where(lane == rr, jnp.sum(a), coll)
        s = pl.ds(g * 16, 16)
        bpv = W1 + W2 * bu_v[s] * bi_v[s]
        return acc + bpv * _softplus16(-coll)

    pos_acc = lax.fori_loop(0, BPW // 16, pos_group,
                            jnp.zeros((16,), jnp.float32))

    def pair_body(i, acc):
        c = i * 2
        fire(c + 1, bufs[1])
        drain(c, bufs[0])
        acc = chunk_neg(c, bufs[0], acc)

        @pl.when(c + 2 < NCH)
        def _():
            fire(c + 2, bufs[0])

        drain(c + 1, bufs[1])
        acc = chunk_neg(c + 1, bufs[1], acc)
        return acc

    neg_acc = lax.fori_loop(0, NCH // 2, pair_body,
                            jnp.zeros((16,), jnp.float32))

    loss_v[pl.ds(0, 16)] = pos_acc + neg_acc * (1.0 / N_NEG)
    pltpu.sync_copy(loss_v, loss_hbm.at[pl.ds(wid * 16, 16)])


LOSS_R = NW * 16 // 128


def _stage_c_body(part_ref, reg_ref, out_ref):
    out_ref[0, 0] = jnp.sum(part_ref[...]) + reg_ref[0, 0]


_stage_c = pl.pallas_call(
    _stage_c_body,
    in_specs=[
        pl.BlockSpec((LOSS_R, 128), lambda: (0, 0)),
        pl.BlockSpec((1, 1), lambda: (0, 0), memory_space=pltpu.SMEM),
    ],
    out_specs=pl.BlockSpec((1, 1), lambda: (0, 0), memory_space=pltpu.SMEM),
    out_shape=jax.ShapeDtypeStruct((1, 1), jnp.float32),
)


def kernel(uid, iid, niid, U, V, W, b, v_feat, mask, beta_u, beta_i):
    item_emb, regs = _stage_a(v_feat, V, U, W, b.reshape(1, FEAT_DIM),
                              mask.reshape(1, RAW_FEAT))
    (parts,) = _stage_b(uid, iid, niid.reshape(-1), U, item_emb,
                        beta_u, beta_i)
    loss = _stage_c(parts.reshape(LOSS_R, 128), regs)
    return loss[0, 0]


# speedup vs baseline: 13.5727x; 1.0640x over previous
import functools

import jax
import jax.numpy as jnp
import numpy as np
from jax import lax
from jax.experimental import pallas as pl
from jax.experimental.pallas import tpu as pltpu
from jax.experimental.pallas import tpu_sc as plsc

USZ = 100000
ISZ = 50000
DIM = 64
FEAT_DIM = 64
RAW_FEAT = 128
BATCH = 4096
N_NEG = 50
W1, W2, W3, W4 = 1e-06, 1.0, 1.0, 1.0
WD1, WD2, WDI = 0.0001, 0.0001, 1.0
EMB = DIM + FEAT_DIM

NW = 32
BPW = BATCH // NW
NPW = BATCH * N_NEG // NW
CHUNK = 128
NCHUNK = NPW // CHUNK


RB = 1000
GA = ISZ // RB
URB = USZ // GA


def _stage_a_body(vf_ref, v_ref, w_ref, b_ref, mask_ref, item_ref, reg_ref):
    i = pl.program_id(0)
    vf = vf_ref[...]
    nrm = jnp.sqrt(jnp.sum(vf * vf, axis=1, keepdims=True))
    vf = vf / jnp.maximum(nrm, 1e-12)
    vf = vf * mask_ref[...]
    feat = lax.dot_general(vf, w_ref[...], (((1,), (1,)), ((), ())),
                           preferred_element_type=jnp.float32)
    feat = feat + b_ref[...]
    v = v_ref[...]
    item_ref[...] = jnp.concatenate([v, feat], axis=1)
    part = (WD1 * WDI) * jnp.sum(v * v)

    @pl.when(i == 0)
    def _():
        w = w_ref[...]
        reg_ref[0, 0] = WD2 * (jnp.sum(w * w) + jnp.sum(b_ref[...] ** 2))

    reg_ref[0, 0] += part


_stage_a = pl.pallas_call(
    _stage_a_body,
    grid=(GA,),
    in_specs=[
        pl.BlockSpec((RB, RAW_FEAT), lambda i: (i, 0)),
        pl.BlockSpec((RB, DIM), lambda i: (i, 0)),
        pl.BlockSpec((FEAT_DIM, RAW_FEAT), lambda i: (0, 0)),
        pl.BlockSpec((1, FEAT_DIM), lambda i: (0, 0)),
        pl.BlockSpec((1, RAW_FEAT), lambda i: (0, 0)),
    ],
    out_specs=[
        pl.BlockSpec((RB, EMB), lambda i: (i, 0)),
        pl.BlockSpec((1, 1), lambda i: (0, 0), memory_space=pltpu.SMEM),
    ],
    out_shape=[
        jax.ShapeDtypeStruct((ISZ, EMB), jnp.float32),
        jax.ShapeDtypeStruct((1, 1), jnp.float32),
    ],
)


def _stage_a2_body(u_ref, reg_ref):
    i = pl.program_id(0)
    u = u_ref[...]
    part = WD1 * jnp.sum(u * u)

    @pl.when(i == 0)
    def _():
        reg_ref[0, 0] = 0.0

    reg_ref[0, 0] += part


_stage_a2 = pl.pallas_call(
    _stage_a2_body,
    grid=(GA,),
    in_specs=[pl.BlockSpec((URB, EMB), lambda i: (i, 0))],
    out_specs=pl.BlockSpec((1, 1), lambda i: (0, 0), memory_space=pltpu.SMEM),
    out_shape=jax.ShapeDtypeStruct((1, 1), jnp.float32),
)


LOG2E = 1.4426950408889634
LN2 = 0.6931471805599453

_sc_mesh = plsc.VectorSubcoreMesh(core_axis_name="c", subcore_axis_name="s")

NR_CH = 4
CH_NEG = NR_CH * N_NEG
NCH = BPW // NR_CH


def _softplus16(x):
    t = jnp.exp(-jnp.abs(x))
    z = t / (2.0 + t)
    z2 = z * z
    s = z * (2.0 + z2 * (2.0 / 3.0 + z2 * (2.0 / 5.0 + z2 * (
        2.0 / 7.0 + z2 * (2.0 / 9.0 + z2 * (2.0 / 11.0))))))
    return jnp.maximum(x, 0.0) + s


@functools.partial(
    pl.kernel,
    mesh=_sc_mesh,
    compiler_params=pltpu.CompilerParams(needs_layout_passes=False),
    out_type=[
        jax.ShapeDtypeStruct((NW * 16,), jnp.float32),
    ],
    scratch_types=[
        pltpu.VMEM((BPW,), jnp.int32),
        pltpu.VMEM((BPW,), jnp.int32),
        pltpu.VMEM((BPW, EMB), jnp.float32),
        pltpu.VMEM((BPW, EMB), jnp.float32),
        pltpu.VMEM((BPW,), jnp.float32),
        pltpu.VMEM((BPW,), jnp.float32),
        pltpu.VMEM((NPW,), jnp.int32),
        pltpu.VMEM((CH_NEG, EMB), jnp.float32),
        pltpu.VMEM((CH_NEG, EMB), jnp.float32),
        pltpu.VMEM((CH_NEG,), jnp.float32),
        pltpu.VMEM((CH_NEG,), jnp.float32),
        pltpu.VMEM((16,), jnp.float32),
        pltpu.SemaphoreType.DMA,
        pltpu.SemaphoreType.DMA,
        pltpu.SemaphoreType.DMA,
    ],
)
def _stage_b(uid_hbm, iid_hbm, niid_hbm, u_hbm, item_hbm, bu_hbm, bi_hbm,
             loss_hbm,
             uid_v, iid_v, urows, prows, bu_v, bi_v,
             nidx_all, nrows_b0, nrows_b1, bin_b0, bin_b1, loss_v,
             sem_pos, sem_b0, sem_b1):
    wid = lax.axis_index("s") * 2 + lax.axis_index("c")
    bbase = wid * BPW
    nbase = wid * NPW
    lane = lax.iota(jnp.int32, 16)
    bufs = ((nrows_b0, bin_b0, sem_b0), (nrows_b1, bin_b1, sem_b1))

    pltpu.sync_copy(uid_hbm.at[pl.ds(bbase, BPW)], uid_v)
    pltpu.sync_copy(iid_hbm.at[pl.ds(bbase, BPW)], iid_v)
    pltpu.async_copy(u_hbm.at[uid_v], urows, sem_pos)
    pltpu.async_copy(item_hbm.at[iid_v], prows, sem_pos)
    pltpu.async_copy(bu_hbm.at[uid_v], bu_v, sem_pos)
    pltpu.async_copy(bi_hbm.at[iid_v], bi_v, sem_pos)
    pltpu.sync_copy(niid_hbm.at[pl.ds(nbase, NPW)], nidx_all)

    def fire(c, buf):
        nrows_b, bin_b, sem = buf
        idx = nidx_all.at[pl.ds(c * CH_NEG, CH_NEG)]
        pltpu.async_copy(item_hbm.at[idx], nrows_b, sem)
        pltpu.async_copy(bi_hbm.at[idx], bin_b, sem)

    def drain(c, buf):
        nrows_b, bin_b, sem = buf
        idx = nidx_all.at[pl.ds(c * CH_NEG, CH_NEG)]
        pltpu.make_async_copy(item_hbm.at[idx], nrows_b, sem).wait()
        pltpu.make_async_copy(bi_hbm.at[idx], bin_b, sem).wait()

    def row_neg(c, row, buf, acc):
        nrows_b, bin_b, _ = buf
        r = c * NR_CH + row
        base = row * N_NEG
        uk = [urows[r, pl.ds(k * 16, 16)] for k in range(8)]
        bu_bc = plsc.load_gather(bu_v, [jnp.zeros((16,), jnp.int32) + r])
        for start, lo in ((0, 0), (16, 0), (32, 0), (34, 14)):
            coll = jnp.zeros((16,), jnp.float32)
            for jj in range(lo, 16):
                j = base + start + jj
                a = nrows_b[j, pl.ds(0, 16)] * uk[0]
                for k in range(1, 8):
                    a = a + nrows_b[j, pl.ds(k * 16, 16)] * uk[k]
                coll = jnp.where(lane == jj, jnp.sum(a), coll)
            binv = plsc.load_gather(bin_b, [base + start + lane])
            con = (W3 + W4 * bu_bc * binv) * _softplus16(coll)
            if lo:
                con = jnp.where(lane >= lo, con, 0.0)
            acc = acc + con
        return acc

    def chunk_neg(c, buf, acc):
        def rb(row, a):
            return row_neg(c, row, buf, a)
        return lax.fori_loop(0, NR_CH, rb, acc)

    fire(0, bufs[0])

    pltpu.make_async_copy(u_hbm.at[uid_v], urows, sem_pos).wait()
    pltpu.make_async_copy(item_hbm.at[iid_v], prows, sem_pos).wait()
    pltpu.make_async_copy(bu_hbm.at[uid_v], bu_v, sem_pos).wait()
    pltpu.make_async_copy(bi_hbm.at[iid_v], bi_v, sem_pos).wait()

    def pos_group(g, acc):
        coll = jnp.zeros((16,), jnp.float32)
        for rr in range(16):
            r = g * 16 + rr
            a = urows[r, pl.ds(0, 16)] * prows[r, pl.ds(0, 16)]
            for k in range(1, 8):
                a = a + urows[r, pl.ds(k * 16, 16)] * prows[r, pl.ds(k * 16, 16)]
            coll = jnp.where(lane == rr, jnp.sum(a), coll)
        s = pl.ds(g * 16, 16)
        bpv = W1 + W2 * bu_v[s] * bi_v[s]
        return acc + bpv * _softplus16(-coll)

    pos_acc = lax.fori_loop(0, BPW // 16, pos_group,
                            jnp.zeros((16,), jnp.float32))

    def pair_body(i, acc):
        c = i * 2
        fire(c + 1, bufs[1])
        drain(c, bufs[0])
        acc = chunk_neg(c, bufs[0], acc)

        @pl.when(c + 2 < NCH)
        def _():
            fire(c + 2, bufs[0])

        drain(c + 1, bufs[1])
        acc = chunk_neg(c + 1, bufs[1], acc)
        return acc

    neg_acc = lax.fori_loop(0, NCH // 2, pair_body,
                            jnp.zeros((16,), jnp.float32))

    loss_v[pl.ds(0, 16)] = pos_acc + neg_acc * (1.0 / N_NEG)
    pltpu.sync_copy(loss_v, loss_hbm.at[pl.ds(wid * 16, 16)])


LOSS_R = NW * 16 // 128


def _stage_c_body(part_ref, reg_ref, regu_ref, out_ref):
    out_ref[0, 0] = jnp.sum(part_ref[...]) + reg_ref[0, 0] + regu_ref[0, 0]


_stage_c = pl.pallas_call(
    _stage_c_body,
    in_specs=[
        pl.BlockSpec((LOSS_R, 128), lambda: (0, 0)),
        pl.BlockSpec((1, 1), lambda: (0, 0), memory_space=pltpu.SMEM),
        pl.BlockSpec((1, 1), lambda: (0, 0), memory_space=pltpu.SMEM),
    ],
    out_specs=pl.BlockSpec((1, 1), lambda: (0, 0), memory_space=pltpu.SMEM),
    out_shape=jax.ShapeDtypeStruct((1, 1), jnp.float32),
)


def kernel(uid, iid, niid, U, V, W, b, v_feat, mask, beta_u, beta_i):
    item_emb, regs = _stage_a(v_feat, V, W, b.reshape(1, FEAT_DIM),
                              mask.reshape(1, RAW_FEAT))
    (parts,) = _stage_b(uid, iid, niid.reshape(-1), U, item_emb,
                        beta_u, beta_i)
    regu = _stage_a2(U)
    loss = _stage_c(parts.reshape(LOSS_R, 128), regs, regu)
    return loss[0, 0]


# speedup vs baseline: 14.6313x; 1.0780x over previous
import functools

import jax
import jax.numpy as jnp
import numpy as np
from jax import lax
from jax.experimental import pallas as pl
from jax.experimental.pallas import tpu as pltpu
from jax.experimental.pallas import tpu_sc as plsc

USZ = 100000
ISZ = 50000
DIM = 64
FEAT_DIM = 64
RAW_FEAT = 128
BATCH = 4096
N_NEG = 50
W1, W2, W3, W4 = 1e-06, 1.0, 1.0, 1.0
WD1, WD2, WDI = 0.0001, 0.0001, 1.0
EMB = DIM + FEAT_DIM

NW = 32
BPW = BATCH // NW
NPW = BATCH * N_NEG // NW
CHUNK = 128
NCHUNK = NPW // CHUNK


RB = 2000
GA = ISZ // RB
URB = USZ // GA


def _stage_a_body(vf_ref, v_ref, w_ref, b_ref, mask_ref, item_ref, reg_ref):
    i = pl.program_id(0)
    vf = vf_ref[...]
    nrm = jnp.sqrt(jnp.sum(vf * vf, axis=1, keepdims=True))
    vf = vf / jnp.maximum(nrm, 1e-12)
    vf = vf * mask_ref[...]
    feat = lax.dot_general(vf, w_ref[...], (((1,), (1,)), ((), ())),
                           preferred_element_type=jnp.float32)
    feat = feat + b_ref[...]
    v = v_ref[...]
    item_ref[...] = jnp.concatenate([v, feat], axis=1)
    part = (WD1 * WDI) * jnp.sum(v * v)

    @pl.when(i == 0)
    def _():
        w = w_ref[...]
        reg_ref[0, 0] = WD2 * (jnp.sum(w * w) + jnp.sum(b_ref[...] ** 2))

    reg_ref[0, 0] += part


_stage_a = pl.pallas_call(
    _stage_a_body,
    grid=(GA,),
    in_specs=[
        pl.BlockSpec((RB, RAW_FEAT), lambda i: (i, 0)),
        pl.BlockSpec((RB, DIM), lambda i: (i, 0)),
        pl.BlockSpec((FEAT_DIM, RAW_FEAT), lambda i: (0, 0)),
        pl.BlockSpec((1, FEAT_DIM), lambda i: (0, 0)),
        pl.BlockSpec((1, RAW_FEAT), lambda i: (0, 0)),
    ],
    out_specs=[
        pl.BlockSpec((RB, EMB), lambda i: (i, 0)),
        pl.BlockSpec((1, 1), lambda i: (0, 0), memory_space=pltpu.SMEM),
    ],
    out_shape=[
        jax.ShapeDtypeStruct((ISZ, EMB), jnp.float32),
        jax.ShapeDtypeStruct((1, 1), jnp.float32),
    ],
)


def _stage_a2_body(u_ref, reg_ref):
    i = pl.program_id(0)
    u = u_ref[...]
    part = WD1 * jnp.sum(u * u)

    @pl.when(i == 0)
    def _():
        reg_ref[0, 0] = 0.0

    reg_ref[0, 0] += part


_stage_a2 = pl.pallas_call(
    _stage_a2_body,
    grid=(GA,),
    in_specs=[pl.BlockSpec((URB, EMB), lambda i: (i, 0))],
    out_specs=pl.BlockSpec((1, 1), lambda i: (0, 0), memory_space=pltpu.SMEM),
    out_shape=jax.ShapeDtypeStruct((1, 1), jnp.float32),
)


LOG2E = 1.4426950408889634
LN2 = 0.6931471805599453

_sc_mesh = plsc.VectorSubcoreMesh(core_axis_name="c", subcore_axis_name="s")

NR_CH = 4
CH_NEG = NR_CH * N_NEG
NCH = BPW // NR_CH


def _softplus16(x):
    t = jnp.exp(-jnp.abs(x))
    z = t / (2.0 + t)
    z2 = z * z
    s = z * (2.0 + z2 * (2.0 / 3.0 + z2 * (2.0 / 5.0 + z2 * (
        2.0 / 7.0 + z2 * (2.0 / 9.0 + z2 * (2.0 / 11.0))))))
    return jnp.maximum(x, 0.0) + s


@functools.partial(
    pl.kernel,
    mesh=_sc_mesh,
    compiler_params=pltpu.CompilerParams(needs_layout_passes=False),
    out_type=[
        jax.ShapeDtypeStruct((NW * 16,), jnp.float32),
    ],
    scratch_types=[
        pltpu.VMEM((BPW,), jnp.int32),
        pltpu.VMEM((BPW,), jnp.int32),
        pltpu.VMEM((BPW, EMB), jnp.float32),
        pltpu.VMEM((BPW, EMB), jnp.float32),
        pltpu.VMEM((BPW,), jnp.float32),
        pltpu.VMEM((BPW,), jnp.float32),
        pltpu.VMEM((NPW,), jnp.int32),
        pltpu.VMEM((CH_NEG, EMB), jnp.float32),
        pltpu.VMEM((CH_NEG, EMB), jnp.float32),
        pltpu.VMEM((CH_NEG,), jnp.float32),
        pltpu.VMEM((CH_NEG,), jnp.float32),
        pltpu.VMEM((16,), jnp.float32),
        pltpu.SemaphoreType.DMA,
        pltpu.SemaphoreType.DMA,
        pltpu.SemaphoreType.DMA,
    ],
)
def _stage_b(uid_hbm, iid_hbm, niid_hbm, u_hbm, item_hbm, bu_hbm, bi_hbm,
             loss_hbm,
             uid_v, iid_v, urows, prows, bu_v, bi_v,
             nidx_all, nrows_b0, nrows_b1, bin_b0, bin_b1, loss_v,
             sem_pos, sem_b0, sem_b1):
    wid = lax.axis_index("s") * 2 + lax.axis_index("c")
    bbase = wid * BPW
    nbase = wid * NPW
    lane = lax.iota(jnp.int32, 16)
    bufs = ((nrows_b0, bin_b0, sem_b0), (nrows_b1, bin_b1, sem_b1))

    pltpu.sync_copy(uid_hbm.at[pl.ds(bbase, BPW)], uid_v)
    pltpu.sync_copy(iid_hbm.at[pl.ds(bbase, BPW)], iid_v)
    pltpu.async_copy(u_hbm.at[uid_v], urows, sem_pos)
    pltpu.async_copy(item_hbm.at[iid_v], prows, sem_pos)
    pltpu.async_copy(bu_hbm.at[uid_v], bu_v, sem_pos)
    pltpu.async_copy(bi_hbm.at[iid_v], bi_v, sem_pos)
    pltpu.sync_copy(niid_hbm.at[pl.ds(nbase, NPW)], nidx_all)

    def fire(c, buf):
        nrows_b, bin_b, sem = buf
        idx = nidx_all.at[pl.ds(c * CH_NEG, CH_NEG)]
        pltpu.async_copy(item_hbm.at[idx], nrows_b, sem)
        pltpu.async_copy(bi_hbm.at[idx], bin_b, sem)

    def drain(c, buf):
        nrows_b, bin_b, sem = buf
        idx = nidx_all.at[pl.ds(c * CH_NEG, CH_NEG)]
        pltpu.make_async_copy(item_hbm.at[idx], nrows_b, sem).wait()
        pltpu.make_async_copy(bi_hbm.at[idx], bin_b, sem).wait()

    def row_neg(c, row, buf, acc):
        nrows_b, bin_b, _ = buf
        r = c * NR_CH + row
        base = row * N_NEG
        uk = [urows[r, pl.ds(k * 16, 16)] for k in range(8)]
        bu_bc = plsc.load_gather(bu_v, [jnp.zeros((16,), jnp.int32) + r])
        for start, lo in ((0, 0), (16, 0), (32, 0), (34, 14)):
            coll = jnp.zeros((16,), jnp.float32)
            for jj in range(lo, 16):
                j = base + start + jj
                a = nrows_b[j, pl.ds(0, 16)] * uk[0]
                for k in range(1, 8):
                    a = a + nrows_b[j, pl.ds(k * 16, 16)] * uk[k]
                coll = jnp.where(lane == jj, jnp.sum(a), coll)
            binv = plsc.load_gather(bin_b, [base + start + lane])
            con = (W3 + W4 * bu_bc * binv) * _softplus16(coll)
            if lo:
                con = jnp.where(lane >= lo, con, 0.0)
            acc = acc + con
        return acc

    def chunk_neg(c, buf, acc):
        def rb(row, a):
            return row_neg(c, row, buf, a)
        return lax.fori_loop(0, NR_CH, rb, acc)

    fire(0, bufs[0])

    pltpu.make_async_copy(u_hbm.at[uid_v], urows, sem_pos).wait()
    pltpu.make_async_copy(item_hbm.at[iid_v], prows, sem_pos).wait()
    pltpu.make_async_copy(bu_hbm.at[uid_v], bu_v, sem_pos).wait()
    pltpu.make_async_copy(bi_hbm.at[iid_v], bi_v, sem_pos).wait()

    def pos_group(g, acc):
        coll = jnp.zeros((16,), jnp.float32)
        for rr in range(16):
            r = g * 16 + rr
            a = urows[r, pl.ds(0, 16)] * prows[r, pl.ds(0, 16)]
            for k in range(1, 8):
                a = a + urows[r, pl.ds(k * 16, 16)] * prows[r, pl.ds(k * 16, 16)]
            coll = jnp.where(lane == rr, jnp.sum(a), coll)
        s = pl.ds(g * 16, 16)
        bpv = W1 + W2 * bu_v[s] * bi_v[s]
        return acc + bpv * _softplus16(-coll)

    pos_acc = lax.fori_loop(0, BPW // 16, pos_group,
                            jnp.zeros((16,), jnp.float32))

    def pair_body(i, acc):
        c = i * 2
        fire(c + 1, bufs[1])
        drain(c, bufs[0])
        acc = chunk_neg(c, bufs[0], acc)

        @pl.when(c + 2 < NCH)
        def _():
            fire(c + 2, bufs[0])

        drain(c + 1, bufs[1])
        acc = chunk_neg(c + 1, bufs[1], acc)
        return acc

    neg_acc = lax.fori_loop(0, NCH // 2, pair_body,
                            jnp.zeros((16,), jnp.float32))

    loss_v[pl.ds(0, 16)] = pos_acc + neg_acc * (1.0 / N_NEG)
    pltpu.sync_copy(loss_v, loss_hbm.at[pl.ds(wid * 16, 16)])


LOSS_R = NW * 16 // 128


def _stage_c_body(part_ref, reg_ref, regu_ref, out_ref):
    out_ref[0, 0] = jnp.sum(part_ref[...]) + reg_ref[0, 0] + regu_ref[0, 0]


_stage_c = pl.pallas_call(
    _stage_c_body,
    in_specs=[
        pl.BlockSpec((LOSS_R, 128), lambda: (0, 0)),
        pl.BlockSpec((1, 1), lambda: (0, 0), memory_space=pltpu.SMEM),
        pl.BlockSpec((1, 1), lambda: (0, 0), memory_space=pltpu.SMEM),
    ],
    out_specs=pl.BlockSpec((1, 1), lambda: (0, 0), memory_space=pltpu.SMEM),
    out_shape=jax.ShapeDtypeStruct((1, 1), jnp.float32),
)


def kernel(uid, iid, niid, U, V, W, b, v_feat, mask, beta_u, beta_i):
    item_emb, regs = _stage_a(v_feat, V, W, b.reshape(1, FEAT_DIM),
                              mask.reshape(1, RAW_FEAT))
    (parts,) = _stage_b(uid, iid, niid.reshape(-1), U, item_emb,
                        beta_u, beta_i)
    regu = _stage_a2(U)
    loss = _stage_c(parts.reshape(LOSS_R, 128), regs, regu)
    return loss[0, 0]


# speedup vs baseline: 15.3205x; 1.0471x over previous
import functools

import jax
import jax.numpy as jnp
import numpy as np
from jax import lax
from jax.experimental import pallas as pl
from jax.experimental.pallas import tpu as pltpu
from jax.experimental.pallas import tpu_sc as plsc

USZ = 100000
ISZ = 50000
DIM = 64
FEAT_DIM = 64
RAW_FEAT = 128
BATCH = 4096
N_NEG = 50
W1, W2, W3, W4 = 1e-06, 1.0, 1.0, 1.0
WD1, WD2, WDI = 0.0001, 0.0001, 1.0
EMB = DIM + FEAT_DIM

NW = 32
BPW = BATCH // NW
NPW = BATCH * N_NEG // NW
CHUNK = 128
NCHUNK = NPW // CHUNK


RB = 5000
GA = ISZ // RB
URB = USZ // GA


def _stage_a_body(vf_ref, v_ref, w_ref, b_ref, mask_ref, item_ref, reg_ref):
    i = pl.program_id(0)
    vf = vf_ref[...]
    nrm = jnp.sqrt(jnp.sum(vf * vf, axis=1, keepdims=True))
    vf = vf / jnp.maximum(nrm, 1e-12)
    vf = vf * mask_ref[...]
    feat = lax.dot_general(vf, w_ref[...], (((1,), (1,)), ((), ())),
                           preferred_element_type=jnp.float32)
    feat = feat + b_ref[...]
    v = v_ref[...]
    item_ref[...] = jnp.concatenate([v, feat], axis=1)
    part = (WD1 * WDI) * jnp.sum(v * v)

    @pl.when(i == 0)
    def _():
        w = w_ref[...]
        reg_ref[0, 0] = WD2 * (jnp.sum(w * w) + jnp.sum(b_ref[...] ** 2))

    reg_ref[0, 0] += part


_stage_a = pl.pallas_call(
    _stage_a_body,
    grid=(GA,),
    in_specs=[
        pl.BlockSpec((RB, RAW_FEAT), lambda i: (i, 0)),
        pl.BlockSpec((RB, DIM), lambda i: (i, 0)),
        pl.BlockSpec((FEAT_DIM, RAW_FEAT), lambda i: (0, 0)),
        pl.BlockSpec((1, FEAT_DIM), lambda i: (0, 0)),
        pl.BlockSpec((1, RAW_FEAT), lambda i: (0, 0)),
    ],
    out_specs=[
        pl.BlockSpec((RB, EMB), lambda i: (i, 0)),
        pl.BlockSpec((1, 1), lambda i: (0, 0), memory_space=pltpu.SMEM),
    ],
    out_shape=[
        jax.ShapeDtypeStruct((ISZ, EMB), jnp.float32),
        jax.ShapeDtypeStruct((1, 1), jnp.float32),
    ],
)


def _stage_a2_body(u_ref, reg_ref):
    i = pl.program_id(0)
    u = u_ref[...]
    part = WD1 * jnp.sum(u * u)

    @pl.when(i == 0)
    def _():
        reg_ref[0, 0] = 0.0

    reg_ref[0, 0] += part


_stage_a2 = pl.pallas_call(
    _stage_a2_body,
    grid=(GA,),
    in_specs=[pl.BlockSpec((URB, EMB), lambda i: (i, 0))],
    out_specs=pl.BlockSpec((1, 1), lambda i: (0, 0), memory_space=pltpu.SMEM),
    out_shape=jax.ShapeDtypeStruct((1, 1), jnp.float32),
)


LOG2E = 1.4426950408889634
LN2 = 0.6931471805599453

_sc_mesh = plsc.VectorSubcoreMesh(core_axis_name="c", subcore_axis_name="s")

NR_CH = 4
CH_NEG = NR_CH * N_NEG
NCH = BPW // NR_CH


def _softplus16(x):
    t = jnp.exp(-jnp.abs(x))
    z = t / (2.0 + t)
    z2 = z * z
    s = z * (2.0 + z2 * (2.0 / 3.0 + z2 * (2.0 / 5.0 + z2 * (
        2.0 / 7.0 + z2 * (2.0 / 9.0 + z2 * (2.0 / 11.0))))))
    return jnp.maximum(x, 0.0) + s


@functools.partial(
    pl.kernel,
    mesh=_sc_mesh,
    compiler_params=pltpu.CompilerParams(needs_layout_passes=False),
    out_type=[
        jax.ShapeDtypeStruct((NW * 16,), jnp.float32),
    ],
    scratch_types=[
        pltpu.VMEM((BPW,), jnp.int32),
        pltpu.VMEM((BPW,), jnp.int32),
        pltpu.VMEM((BPW, EMB), jnp.float32),
        pltpu.VMEM((BPW, EMB), jnp.float32),
        pltpu.VMEM((BPW,), jnp.float32),
        pltpu.VMEM((BPW,), jnp.float32),
        pltpu.VMEM((NPW,), jnp.int32),
        pltpu.VMEM((CH_NEG, EMB), jnp.float32),
        pltpu.VMEM((CH_NEG, EMB), jnp.float32),
        pltpu.VMEM((CH_NEG,), jnp.float32),
        pltpu.VMEM((CH_NEG,), jnp.float32),
        pltpu.VMEM((16,), jnp.float32),
        pltpu.SemaphoreType.DMA,
        pltpu.SemaphoreType.DMA,
        pltpu.SemaphoreType.DMA,
    ],
)
def _stage_b(uid_hbm, iid_hbm, niid_hbm, u_hbm, item_hbm, bu_hbm, bi_hbm,
             loss_hbm,
             uid_v, iid_v, urows, prows, bu_v, bi_v,
             nidx_all, nrows_b0, nrows_b1, bin_b0, bin_b1, loss_v,
             sem_pos, sem_b0, sem_b1):
    wid = lax.axis_index("s") * 2 + lax.axis_index("c")
    bbase = wid * BPW
    nbase = wid * NPW
    lane = lax.iota(jnp.int32, 16)
    bufs = ((nrows_b0, bin_b0, sem_b0), (nrows_b1, bin_b1, sem_b1))

    pltpu.sync_copy(uid_hbm.at[pl.ds(bbase, BPW)], uid_v)
    pltpu.sync_copy(iid_hbm.at[pl.ds(bbase, BPW)], iid_v)
    pltpu.async_copy(u_hbm.at[uid_v], urows, sem_pos)
    pltpu.async_copy(item_hbm.at[iid_v], prows, sem_pos)
    pltpu.async_copy(bu_hbm.at[uid_v], bu_v, sem_pos)
    pltpu.async_copy(bi_hbm.at[iid_v], bi_v, sem_pos)
    pltpu.sync_copy(niid_hbm.at[pl.ds(nbase, NPW)], nidx_all)

    def fire(c, buf):
        nrows_b, bin_b, sem = buf
        idx = nidx_all.at[pl.ds(c * CH_NEG, CH_NEG)]
        pltpu.async_copy(item_hbm.at[idx], nrows_b, sem)
        pltpu.async_copy(bi_hbm.at[idx], bin_b, sem)

    def drain(c, buf):
        nrows_b, bin_b, sem = buf
        idx = nidx_all.at[pl.ds(c * CH_NEG, CH_NEG)]
        pltpu.make_async_copy(item_hbm.at[idx], nrows_b, sem).wait()
        pltpu.make_async_copy(bi_hbm.at[idx], bin_b, sem).wait()

    def row_neg(c, row, buf, acc):
        nrows_b, bin_b, _ = buf
        r = c * NR_CH + row
        base = row * N_NEG
        uk = [urows[r, pl.ds(k * 16, 16)] for k in range(8)]
        bu_bc = plsc.load_gather(bu_v, [jnp.zeros((16,), jnp.int32) + r])
        for start, lo in ((0, 0), (16, 0), (32, 0), (34, 14)):
            coll = jnp.zeros((16,), jnp.float32)
            for jj in range(lo, 16):
                j = base + start + jj
                a = nrows_b[j, pl.ds(0, 16)] * uk[0]
                for k in range(1, 8):
                    a = a + nrows_b[j, pl.ds(k * 16, 16)] * uk[k]
                coll = jnp.where(lane == jj, jnp.sum(a), coll)
            binv = plsc.load_gather(bin_b, [base + start + lane])
            con = (W3 + W4 * bu_bc * binv) * _softplus16(coll)
            if lo:
                con = jnp.where(lane >= lo, con, 0.0)
            acc = acc + con
        return acc

    def chunk_neg(c, buf, acc):
        def rb(row, a):
            return row_neg(c, row, buf, a)
        return lax.fori_loop(0, NR_CH, rb, acc)

    fire(0, bufs[0])

    pltpu.make_async_copy(u_hbm.at[uid_v], urows, sem_pos).wait()
    pltpu.make_async_copy(item_hbm.at[iid_v], prows, sem_pos).wait()
    pltpu.make_async_copy(bu_hbm.at[uid_v], bu_v, sem_pos).wait()
    pltpu.make_async_copy(bi_hbm.at[iid_v], bi_v, sem_pos).wait()

    def pos_group(g, acc):
        coll = jnp.zeros((16,), jnp.float32)
        for rr in range(16):
            r = g * 16 + rr
            a = urows[r, pl.ds(0, 16)] * prows[r, pl.ds(0, 16)]
            for k in range(1, 8):
                a = a + urows[r, pl.ds(k * 16, 16)] * prows[r, pl.ds(k * 16, 16)]
            coll = jnp.where(lane == rr, jnp.sum(a), coll)
        s = pl.ds(g * 16, 16)
        bpv = W1 + W2 * bu_v[s] * bi_v[s]
        return acc + bpv * _softplus16(-coll)

    pos_acc = lax.fori_loop(0, BPW // 16, pos_group,
                            jnp.zeros((16,), jnp.float32))

    def pair_body(i, acc):
        c = i * 2
        fire(c + 1, bufs[1])
        drain(c, bufs[0])
        acc = chunk_neg(c, bufs[0], acc)

        @pl.when(c + 2 < NCH)
        def _():
            fire(c + 2, bufs[0])

        drain(c + 1, bufs[1])
        acc = chunk_neg(c + 1, bufs[1], acc)
        return acc

    neg_acc = lax.fori_loop(0, NCH // 2, pair_body,
                            jnp.zeros((16,), jnp.float32))

    loss_v[pl.ds(0, 16)] = pos_acc + neg_acc * (1.0 / N_NEG)
    pltpu.sync_copy(loss_v, loss_hbm.at[pl.ds(wid * 16, 16)])


LOSS_R = NW * 16 // 128


def _stage_c_body(part_ref, reg_ref, regu_ref, out_ref):
    out_ref[0, 0] = jnp.sum(part_ref[...]) + reg_ref[0, 0] + regu_ref[0, 0]


_stage_c = pl.pallas_call(
    _stage_c_body,
    in_specs=[
        pl.BlockSpec((LOSS_R, 128), lambda: (0, 0)),
        pl.BlockSpec((1, 1), lambda: (0, 0), memory_space=pltpu.SMEM),
        pl.BlockSpec((1, 1), lambda: (0, 0), memory_space=pltpu.SMEM),
    ],
    out_specs=pl.BlockSpec((1, 1), lambda: (0, 0), memory_space=pltpu.SMEM),
    out_shape=jax.ShapeDtypeStruct((1, 1), jnp.float32),
)


def kernel(uid, iid, niid, U, V, W, b, v_feat, mask, beta_u, beta_i):
    item_emb, regs = _stage_a(v_feat, V, W, b.reshape(1, FEAT_DIM),
                              mask.reshape(1, RAW_FEAT))
    (parts,) = _stage_b(uid, iid, niid.reshape(-1), U, item_emb,
                        beta_u, beta_i)
    regu = _stage_a2(U)
    loss = _stage_c(parts.reshape(LOSS_R, 128), regs, regu)
    return loss[0, 0]
